# baseline (speedup 1.0000x reference)
import jax
import jax.numpy as jnp
from jax.experimental import pallas as pl
from jax.experimental.pallas import tpu as pltpu

_EPS = 1e-6
_G = 128
_FP8_MAX = 448.0
_BM = 256


def _rms_quant_body(x_ref, res_ref, w_ref, q_ref, s_ref, h_ref):
    h = x_ref[...] + res_ref[...]
    h_ref[...] = h
    n = h.shape[1]
    ss = jnp.sum(h * h, axis=1, keepdims=True)
    inv_rms = jax.lax.rsqrt(ss * (1.0 / n) + _EPS)
    y = h * inv_rms * w_ref[...]
    scales = []
    for g in range(n // _G):
        yg = y[:, g * _G:(g + 1) * _G]
        amax = jnp.max(jnp.abs(yg), axis=1, keepdims=True)
        s = jnp.maximum(amax, 1e-10) * (1.0 / _FP8_MAX)
        q = jnp.clip(yg / s, -_FP8_MAX, _FP8_MAX)
        q_ref[:, g * _G:(g + 1) * _G] = q.astype(jnp.float8_e4m3fn)
        scales.append(s)
    s_ref[...] = jnp.concatenate(scales, axis=1)


def kernel(x, res, weight):
    M, N = x.shape
    NG = N // _G
    w2 = weight.reshape(1, N)

    steps = M // _BM
    half = steps // 2

    q, s, h = pl.pallas_call(
        _rms_quant_body,
        grid=(2, half),
        in_specs=[
            pl.BlockSpec((_BM, N), lambda c, i: (c * half + i, 0)),
            pl.BlockSpec((_BM, N), lambda c, i: (c * half + i, 0)),
            pl.BlockSpec((1, N), lambda c, i: (0, 0)),
        ],
        out_specs=[
            pl.BlockSpec((_BM, N), lambda c, i: (c * half + i, 0)),
            pl.BlockSpec((_BM, NG), lambda c, i: (c * half + i, 0)),
            pl.BlockSpec((_BM, N), lambda c, i: (c * half + i, 0)),
        ],
        out_shape=[
            jax.ShapeDtypeStruct((M, N), jnp.float8_e4m3fn),
            jax.ShapeDtypeStruct((M, NG), jnp.float32),
            jax.ShapeDtypeStruct((M, N), jnp.float32),
        ],
        compiler_params=pltpu.CompilerParams(
            dimension_semantics=("parallel", "arbitrary"),
        ),
    )(x, res, w2)

    return q, s, h


# speedup vs baseline: 1.0176x; 1.0176x over previous
import jax
import jax.numpy as jnp
from jax.experimental import pallas as pl
from jax.experimental.pallas import tpu as pltpu

_EPS = 1e-6
_G = 128
_FP8_MAX = 448.0
_BM = 512


def _rms_quant_body(x_ref, res_ref, w_ref, q_ref, s_ref, h_ref):
    h = x_ref[...] + res_ref[...]
    h_ref[...] = h
    n = h.shape[1]
    ss = jnp.sum(h * h, axis=1, keepdims=True)
    inv_rms = jax.lax.rsqrt(ss * (1.0 / n) + _EPS)
    y = h * inv_rms * w_ref[...]
    scales = []
    for g in range(n // _G):
        yg = y[:, g * _G:(g + 1) * _G]
        amax = jnp.max(jnp.abs(yg), axis=1, keepdims=True)
        s = jnp.maximum(amax, 1e-10) * (1.0 / _FP8_MAX)
        q = jnp.clip(yg / s, -_FP8_MAX, _FP8_MAX)
        q_ref[:, g * _G:(g + 1) * _G] = q.astype(jnp.float8_e4m3fn)
        scales.append(s)
    s_ref[...] = jnp.concatenate(scales, axis=1)


def kernel(x, res, weight):
    M, N = x.shape
    NG = N // _G
    w2 = weight.reshape(1, N)

    steps = M // _BM
    half = steps // 2

    q, s, h = pl.pallas_call(
        _rms_quant_body,
        grid=(2, half),
        in_specs=[
            pl.BlockSpec((_BM, N), lambda c, i: (c * half + i, 0)),
            pl.BlockSpec((_BM, N), lambda c, i: (c * half + i, 0)),
            pl.BlockSpec((1, N), lambda c, i: (0, 0)),
        ],
        out_specs=[
            pl.BlockSpec((_BM, N), lambda c, i: (c * half + i, 0)),
            pl.BlockSpec((_BM, NG), lambda c, i: (c * half + i, 0)),
            pl.BlockSpec((_BM, N), lambda c, i: (c * half + i, 0)),
        ],
        out_shape=[
            jax.ShapeDtypeStruct((M, N), jnp.float8_e4m3fn),
            jax.ShapeDtypeStruct((M, NG), jnp.float32),
            jax.ShapeDtypeStruct((M, N), jnp.float32),
        ],
        compiler_params=pltpu.CompilerParams(
            dimension_semantics=("parallel", "arbitrary"),
        ),
    )(x, res, w2)

    return q, s, h
